# baseline (speedup 1.0000x reference)
import functools
import math

import jax
import jax.numpy as jnp
from jax import lax
from jax.experimental import pallas as pl
from jax.experimental.pallas import tpu as pltpu
from jax.experimental.pallas import tpu_sc as plsc

INPUT_DIM = 64
NUM_PROJS = 7
NUM_BUCKETS = 1 << NUM_PROJS
BLOCK_SIZE = 256
SAMPLE_SIZE = 256
N_SEQ = 8192
NUM_BLOCKS = N_SEQ // BLOCK_SIZE
RANK_CHUNK = 256


def _hash_rank_body(q_ref, k_ref, pd_ref, posq_ref, posk_ref):
    pd = pd_ref[...]
    lane = lax.broadcasted_iota(jnp.int32, (N_SEQ, NUM_BUCKETS), 1)
    enc = jnp.where(lane < NUM_PROJS, 1 << jnp.minimum(lane, NUM_PROJS - 1), 0)
    r = lax.broadcasted_iota(jnp.int32, (RANK_CHUNK, RANK_CHUNK), 0)
    c = lax.broadcasted_iota(jnp.int32, (RANK_CHUNK, RANK_CHUNK), 1)
    L_incl = (c <= r).astype(jnp.float32)
    br = lax.broadcasted_iota(jnp.int32, (NUM_BUCKETS, NUM_BUCKETS), 0)
    bc = lax.broadcasted_iota(jnp.int32, (NUM_BUCKETS, NUM_BUCKETS), 1)
    SU = (br < bc).astype(jnp.float32)

    def rank_of(x):
        proj = jax.lax.dot_general(x, pd, (((1,), (0,)), ((), ())),
                                   preferred_element_type=jnp.float32)
        bits = jnp.where((proj > 0) & (lane < NUM_PROJS), enc, 0)
        binv = jnp.sum(bits, axis=1, keepdims=True)
        h = binv ^ (binv >> 1)
        oh = (h == lane).astype(jnp.float32)
        hist = jnp.sum(oh, axis=0, keepdims=True)
        bs = jax.lax.dot_general(hist, SU, (((1,), (0,)), ((), ())),
                                 preferred_element_type=jnp.float32)

        def chunk(i, carry):
            ohc = oh[i * RANK_CHUNK:(i + 1) * RANK_CHUNK, :]
            incl = jax.lax.dot_general(L_incl, ohc, (((1,), (0,)), ((), ())),
                                       preferred_element_type=jnp.float32)
            posc = jnp.sum(ohc * (bs + carry + incl), axis=1) - 1.0
            carry = carry + jnp.sum(ohc, axis=0, keepdims=True)
            return posc.astype(jnp.int32), carry

        carry = jnp.zeros((1, NUM_BUCKETS), jnp.float32)
        pieces = []
        for i in range(N_SEQ // RANK_CHUNK):
            posc, carry = chunk(i, carry)
            pieces.append(posc)
        return jnp.concatenate(pieces, axis=0)

    posq_ref[0, 0] = rank_of(q_ref[0])
    posk_ref[0, 0] = rank_of(k_ref[0])


def _hash_rank(q2, k2, proj_pad):
    BH = q2.shape[0]
    qspec = pl.BlockSpec((1, N_SEQ, INPUT_DIM), lambda i: (i, 0, 0))
    pspec = pl.BlockSpec((INPUT_DIM, NUM_BUCKETS), lambda i: (0, 0))
    ospec = pl.BlockSpec((1, 1, N_SEQ), lambda i: (i, 0, 0))
    pos_q, pos_k = pl.pallas_call(
        _hash_rank_body,
        grid=(BH,),
        in_specs=[qspec, qspec, pspec],
        out_specs=[ospec, ospec],
        out_shape=[jax.ShapeDtypeStruct((BH, 1, N_SEQ), jnp.int32),
                   jax.ShapeDtypeStruct((BH, 1, N_SEQ), jnp.int32)],
    )(q2, k2, proj_pad)
    return pos_q.reshape(BH, N_SEQ), pos_k.reshape(BH, N_SEQ)


def _make_invert_kernel(BH):
    mesh = plsc.VectorSubcoreMesh(core_axis_name="c", subcore_axis_name="s")

    @functools.partial(
        pl.kernel,
        out_type=[jax.ShapeDtypeStruct((BH, N_SEQ), jnp.int32),
                  jax.ShapeDtypeStruct((BH, N_SEQ), jnp.int32),
                  jax.ShapeDtypeStruct((BH, SAMPLE_SIZE), jnp.int32)],
        mesh=mesh,
        scratch_types=[pltpu.VMEM((N_SEQ,), jnp.int32),
                       pltpu.VMEM((N_SEQ,), jnp.int32),
                       pltpu.VMEM((SAMPLE_SIZE,), jnp.int32),
                       pltpu.VMEM((SAMPLE_SIZE,), jnp.int32)],
        compiler_params=pltpu.CompilerParams(needs_layout_passes=False),
    )
    def invert(posq_hbm, posk_hbm, samp_hbm, qsort_hbm, ksort_hbm, sidx_hbm,
               pos_v, srt_v, samp_v, sidx_v):
        wid = lax.axis_index("s") * 2 + lax.axis_index("c")
        iota16 = lax.iota(jnp.int32, 16)

        def invert_row(pos_row_hbm, out_row_hbm):
            pltpu.sync_copy(pos_row_hbm, pos_v)

            def step(j, _):
                idx = pos_v[pl.ds(j * 16, 16)]
                plsc.store_scatter(srt_v, [idx], j * 16 + iota16)
                return 0

            lax.fori_loop(0, N_SEQ // 16, step, 0)
            pltpu.sync_copy(srt_v, out_row_hbm)

        invert_row(posq_hbm.at[wid], qsort_hbm.at[wid])
        invert_row(posk_hbm.at[wid], ksort_hbm.at[wid])
        pltpu.sync_copy(samp_hbm.at[wid], samp_v)

        def gstep(j, _):
            sv = samp_v[pl.ds(j * 16, 16)]
            sidx_v[pl.ds(j * 16, 16)] = plsc.load_gather(srt_v, [sv])
            return 0

        lax.fori_loop(0, SAMPLE_SIZE // 16, gstep, 0)
        pltpu.sync_copy(sidx_v, sidx_hbm.at[wid])

    return invert


def _attn_body(q_ref, kb_ref, vb_ref, ks_ref, vs_ref, samp_ref, out_ref):
    nb = pl.program_id(1)
    scale = INPUT_DIM ** (-0.5)
    qb = q_ref[0, 0]
    kb = kb_ref[0, 0]
    vb = vb_ref[0, 0]
    ks = ks_ref[0]
    vs = vs_ref[0]
    samp = samp_ref[0, 0]

    s1 = jax.lax.dot_general(qb, kb, (((1,), (1,)), ((), ())),
                             preferred_element_type=jnp.float32) * scale
    m1 = jnp.max(s1, axis=1, keepdims=True)
    p1 = jnp.exp(s1 - m1)
    l1 = jnp.sum(p1, axis=1, keepdims=True)
    a1 = jax.lax.dot_general(p1, vb, (((1,), (0,)), ((), ())),
                             preferred_element_type=jnp.float32)
    lse1 = m1 + jnp.log(l1)

    s2 = jax.lax.dot_general(qb, ks, (((1,), (1,)), ((), ())),
                             preferred_element_type=jnp.float32) * scale
    blk_of_samp = samp // BLOCK_SIZE
    neg = jnp.float32(jnp.finfo(jnp.float32).min)
    bias = jnp.where(blk_of_samp == nb, neg, jnp.float32(0.0))[None, :]
    s2 = s2 + bias
    m2 = jnp.max(s2, axis=1, keepdims=True)
    p2 = jnp.exp(s2 - m2)
    l2 = jnp.sum(p2, axis=1, keepdims=True)
    a2 = jax.lax.dot_general(p2, vs, (((1,), (0,)), ((), ())),
                             preferred_element_type=jnp.float32)
    lse2 = m2 + jnp.log(l2) + jnp.float32(math.log(N_SEQ / SAMPLE_SIZE))

    c = jax.nn.sigmoid(lse1 - lse2)
    out = c * (a1 / l1) + (1.0 - c) * (a2 / l2)
    out_ref[0, 0] = out


def _fused_attention(q_sorted, k_sorted, v_sorted, k_sub, v_sub, samp):
    BH, N, D = q_sorted.shape
    nb = NUM_BLOCKS
    qs4 = q_sorted.reshape(BH, nb, BLOCK_SIZE, D)
    ks4 = k_sorted.reshape(BH, nb, BLOCK_SIZE, D)
    vs4 = v_sorted.reshape(BH, nb, BLOCK_SIZE, D)
    grid = (BH, nb)
    blk = pl.BlockSpec((1, 1, BLOCK_SIZE, D), lambda i, j: (i, j, 0, 0))
    sub = pl.BlockSpec((1, SAMPLE_SIZE, D), lambda i, j: (i, 0, 0))
    sspec = pl.BlockSpec((1, 1, SAMPLE_SIZE), lambda i, j: (i, 0, 0))
    out = pl.pallas_call(
        _attn_body,
        grid=grid,
        in_specs=[blk, blk, blk, sub, sub, sspec],
        out_specs=blk,
        out_shape=jax.ShapeDtypeStruct((BH, nb, BLOCK_SIZE, D), jnp.float32),
    )(qs4, ks4, vs4, k_sub, v_sub, samp)
    return out.reshape(BH, N, D)


def kernel(query, key, value, proj_dir, sampled_set):
    B, H, N, D = query.shape
    BH = B * H
    q2 = query.reshape(BH, N, D)
    k2 = key.reshape(BH, N, D)
    v2 = value.reshape(BH, N, D)
    samp2 = sampled_set.reshape(BH, SAMPLE_SIZE)

    proj_pad = jnp.zeros((INPUT_DIM, NUM_BUCKETS), jnp.float32)
    proj_pad = proj_pad.at[:, :NUM_PROJS].set(proj_dir[:INPUT_DIM])

    pos_q, pos_k = _hash_rank(q2, k2, proj_pad)
    q_sort, k_sort, sidx = _make_invert_kernel(BH)(pos_q, pos_k, samp2)

    q_sorted = jnp.take_along_axis(q2, q_sort[..., None], axis=1)
    k_sorted = jnp.take_along_axis(k2, k_sort[..., None], axis=1)
    v_sorted = jnp.take_along_axis(v2, k_sort[..., None], axis=1)
    k_sub = jnp.take_along_axis(k2, sidx[..., None], axis=1)
    v_sub = jnp.take_along_axis(v2, sidx[..., None], axis=1)

    merged = _fused_attention(q_sorted, k_sorted, v_sorted, k_sub, v_sub,
                              samp2.reshape(BH, 1, SAMPLE_SIZE))

    out = jnp.take_along_axis(merged, pos_q[..., None], axis=1)
    return out.reshape(B, H, N, D)
